# Initial kernel scaffold; baseline (speedup 1.0000x reference)
import jax
import jax.numpy as jnp
from jax.experimental import pallas as pl


def kernel(x, edge_index, params):
    raise NotImplementedError("write your pallas kernel here")


# speedup vs baseline: 3.5671x; 3.5671x over previous
import functools

import jax
import jax.numpy as jnp
from jax import lax
from jax.experimental import pallas as pl
from jax.experimental.pallas import tpu as pltpu
from jax.experimental.pallas import tpu_sc as plsc

N = 10000
E = 320000
H = 200
NBLK = 10
RB = N // NBLK

NC = 2
NS = 16
NW = NC * NS
UNITS = 128
UPW = UNITS // NW
NPU = 80
WY = 256
NP_PAD = UNITS * NPU
CHUNK = 128


def _sc_stats_kernel(width):
    NF = width // 16
    mesh = plsc.VectorSubcoreMesh(
        core_axis_name="c", subcore_axis_name="s",
        num_cores=NC, num_subcores=NS)

    @functools.partial(
        pl.kernel,
        out_type=(
            jax.ShapeDtypeStruct((NP_PAD, width), jnp.float32),
            jax.ShapeDtypeStruct((NP_PAD, width), jnp.float32),
            jax.ShapeDtypeStruct((NP_PAD, width), jnp.float32),
            jax.ShapeDtypeStruct((NP_PAD, width), jnp.float32),
            jax.ShapeDtypeStruct((NP_PAD, 16), jnp.float32),
        ),
        mesh=mesh,
        scratch_types=[
            pltpu.VMEM((CHUNK,), jnp.int32),
            pltpu.VMEM((CHUNK, width), jnp.float32),
            pltpu.VMEM((CHUNK + 16,), jnp.int32),
            pltpu.VMEM((144,), jnp.int32),
            pltpu.VMEM((NPU, width), jnp.float32),
            pltpu.VMEM((NPU, width), jnp.float32),
            pltpu.VMEM((NPU, width), jnp.float32),
            pltpu.VMEM((NPU, width), jnp.float32),
            pltpu.VMEM((NPU, 16), jnp.float32),
            pltpu.SemaphoreType.DMA,
        ],
    )
    def k(h_hbm, src_hbm, dst_hbm, ub_hbm,
          o_s, o_q, o_mn, o_mx, o_c,
          idx_v, msgs, dst_sm, ub_sm, a_s, a_q, a_mn, a_mx, a_c, sem):
        wid = lax.axis_index("s") * NC + lax.axis_index("c")
        pltpu.sync_copy(ub_hbm, ub_sm)

        def _at(ref, i):
            return ref[pl.ds(i, 16)][0]
        zeros = jnp.zeros((16,), jnp.float32)
        pinf = jnp.full((16,), jnp.inf, jnp.float32)
        ninf = jnp.full((16,), -jnp.inf, jnp.float32)
        ones = jnp.ones((16,), jnp.float32)
        for ui in range(UPW):
            u = ui * NW + wid
            base = u * NPU
            ubv = ub_sm[pl.ds(u, 16)]
            e0 = ubv[0]
            e1 = ubv[1]

            def init_row(r, carry):
                for v in range(NF):
                    sl = pl.ds(v * 16, 16)
                    a_s[r, sl] = zeros
                    a_q[r, sl] = zeros
                    a_mn[r, sl] = pinf
                    a_mx[r, sl] = ninf
                a_c[r, pl.ds(0, 16)] = zeros
                return carry
            lax.fori_loop(0, NPU, init_row, 0)

            estart = e0 - lax.rem(e0, 8)
            nch = lax.div(e1 - estart + (CHUNK - 1), CHUNK)

            def chunk_body(c, carry):
                e = pl.multiple_of(estart + c * CHUNK, 8)
                pltpu.sync_copy(src_hbm.at[pl.ds(e, CHUNK)], idx_v)
                pltpu.sync_copy(dst_hbm.at[pl.ds(e, CHUNK)],
                                dst_sm.at[pl.ds(0, CHUNK)])
                pltpu.async_copy(h_hbm.at[idx_v], msgs, sem).wait()
                i_lo = jnp.maximum(e0 - e, 0)
                i_hi = jnp.minimum(e1 - e, CHUNK)

                def edge_body(i, ecarry):
                    d = _at(dst_sm, i) - base
                    for v in range(NF):
                        sl = pl.ds(v * 16, 16)
                        m = msgs[i, sl]
                        plsc.addupdate(a_s.at[d, sl], m)
                        plsc.addupdate(a_q.at[d, sl], m * m)
                        a_mn[d, sl] = jnp.minimum(a_mn[d, sl], m)
                        a_mx[d, sl] = jnp.maximum(a_mx[d, sl], m)
                    plsc.addupdate(a_c.at[d, pl.ds(0, 16)], ones)
                    return ecarry
                lax.fori_loop(i_lo, i_hi, edge_body, 0)
                return carry
            lax.fori_loop(0, nch, chunk_body, 0)

            pltpu.sync_copy(a_s, o_s.at[pl.ds(base, NPU)])
            pltpu.sync_copy(a_q, o_q.at[pl.ds(base, NPU)])
            pltpu.sync_copy(a_mn, o_mn.at[pl.ds(base, NPU)])
            pltpu.sync_copy(a_mx, o_mx.at[pl.ds(base, NPU)])
            pltpu.sync_copy(a_c, o_c.at[pl.ds(base, NPU)])
    return k


def _tc_layer_kernel(c_in, width):
    def body(s_ref, q_ref, mn_ref, mx_ref, c_ref, h_ref,
             wp_ref, bp_ref, wl_ref, bl_ref, wr_ref, br_ref,
             y_ref, ps_ref, pq_ref):
        cnt = c_ref[:, 0:1]
        denom = jnp.maximum(cnt, 1.0)
        mean = s_ref[:, :c_in] / denom
        mean2 = q_ref[:, :c_in] / denom
        var = mean2 - mean * mean
        std = jnp.sqrt(jnp.clip(var, 1e-5, None))
        has = cnt > 0.0
        mn = jnp.where(has, mn_ref[:, :c_in], 0.0)
        mx = jnp.where(has, mx_ref[:, :c_in], 0.0)
        wp = wp_ref[...]
        f32 = jnp.float32
        aggr = bp_ref[...]
        aggr = aggr + jnp.dot(mean, wp[0:c_in], preferred_element_type=f32)
        aggr = aggr + jnp.dot(mn, wp[c_in:2 * c_in], preferred_element_type=f32)
        aggr = aggr + jnp.dot(mx, wp[2 * c_in:3 * c_in], preferred_element_type=f32)
        aggr = aggr + jnp.dot(std, wp[3 * c_in:4 * c_in], preferred_element_type=f32)
        aggr = aggr + jnp.dot(var, wp[4 * c_in:5 * c_in], preferred_element_type=f32)
        out = (jnp.dot(aggr, wl_ref[...], preferred_element_type=f32) + bl_ref[...]
               + jnp.dot(h_ref[:, :c_in], wr_ref[...], preferred_element_type=f32)
               + br_ref[...])
        nrm = jnp.sqrt(jnp.sum(out * out, axis=1, keepdims=True))
        out = out / jnp.maximum(nrm, 1e-12)
        y = jnp.maximum(out, 0.0)
        y_ref[:, 0:H] = y
        y_ref[:, H:WY] = jnp.zeros((RB, WY - H), jnp.float32)
        ps_ref[0, 0, :] = jnp.sum(y, axis=0)
        pq_ref[0, 0, :] = jnp.sum(y * y, axis=0)

    stat_spec = pl.BlockSpec((RB, width), lambda b: (b, 0))
    full = lambda shape: pl.BlockSpec(shape, lambda b: (0, 0))
    return pl.pallas_call(
        body,
        grid=(NBLK,),
        in_specs=[
            stat_spec, stat_spec, stat_spec, stat_spec,
            pl.BlockSpec((RB, 16), lambda b: (b, 0)),
            pl.BlockSpec((RB, width), lambda b: (b, 0)),
            full((5 * c_in, H)), full((1, H)),
            full((H, H)), full((1, H)),
            full((c_in, H)), full((1, H)),
        ],
        out_specs=[
            pl.BlockSpec((RB, WY), lambda b: (b, 0)),
            pl.BlockSpec((1, 1, H), lambda b: (b, 0, 0)),
            pl.BlockSpec((1, 1, H), lambda b: (b, 0, 0)),
        ],
        out_shape=[
            jax.ShapeDtypeStruct((N, WY), jnp.float32),
            jax.ShapeDtypeStruct((NBLK, 1, H), jnp.float32),
            jax.ShapeDtypeStruct((NBLK, 1, H), jnp.float32),
        ],
    )


def _tc_bn_kernel():
    def body(y_ref, ps_ref, pq_ref, g_ref, b_ref, o_ref):
        mu = jnp.sum(ps_ref[...], axis=0) / N
        varb = jnp.sum(pq_ref[...], axis=0) / N - mu * mu
        scale = g_ref[...] / jnp.sqrt(varb + 1e-5)
        o_ref[:, 0:H] = (y_ref[:, 0:H] - mu) * scale + b_ref[...]
        o_ref[:, H:WY] = jnp.zeros((RB, WY - H), jnp.float32)

    full = lambda shape: pl.BlockSpec(shape, lambda b: (0, 0))
    return pl.pallas_call(
        body,
        grid=(NBLK,),
        in_specs=[
            pl.BlockSpec((RB, WY), lambda b: (b, 0)),
            pl.BlockSpec((NBLK, 1, H), lambda b: (0, 0, 0)),
            pl.BlockSpec((NBLK, 1, H), lambda b: (0, 0, 0)),
            full((1, H)), full((1, H)),
        ],
        out_specs=pl.BlockSpec((RB, WY), lambda b: (b, 0)),
        out_shape=jax.ShapeDtypeStruct((N, WY), jnp.float32),
    )


def kernel(x, edge_index, params):
    src = edge_index[0]
    dst = edge_index[1]
    order = jnp.argsort(dst)
    dst_s = dst[order]
    src_s = src[order]
    ub = jnp.searchsorted(
        dst_s, (jnp.arange(144, dtype=jnp.int32) * NPU).clip(0, N),
        side="left").astype(jnp.int32)
    pad = jnp.zeros((CHUNK + 8,), jnp.int32)
    srcp = jnp.concatenate([src_s, pad])
    dstp = jnp.concatenate([dst_s, pad])

    sc128 = _sc_stats_kernel(128)
    sc256 = _sc_stats_kernel(WY)
    bn = _tc_bn_kernel()

    h = jnp.pad(x, ((0, 0), (0, 125)))
    for i, p in enumerate(params):
        c_in = 3 if i == 0 else H
        width = 128 if i == 0 else WY
        s, q, mn, mx, cnt = (sc128 if i == 0 else sc256)(h, srcp, dstp, ub)
        k1 = _tc_layer_kernel(c_in, width)
        y, ps, pq = k1(
            s[:N], q[:N], mn[:N], mx[:N], cnt[:N], h,
            p["W_proj"], p["b_proj"].reshape(1, H),
            p["W_l"], p["b_l"].reshape(1, H),
            p["W_r"], p["b_r"].reshape(1, H))
        h = bn(y, ps, pq, p["gamma"].reshape(1, H), p["beta"].reshape(1, H))
    return h[:, :H]


# speedup vs baseline: 6.8499x; 1.9203x over previous
import functools

import jax
import jax.numpy as jnp
from jax import lax
from jax.experimental import pallas as pl
from jax.experimental.pallas import tpu as pltpu
from jax.experimental.pallas import tpu_sc as plsc

N = 10000
E = 320000
H = 200
NBLK = 10
RB = N // NBLK

NC = 2
NS = 16
NW = NC * NS
UNITS = 128
UPW = UNITS // NW
NPU = 80
WY = 256
NP_PAD = UNITS * NPU
CHUNK = 128


def _sc_stats_kernel(width):
    NF = width // 16
    FC = NF // 8
    mesh = plsc.VectorSubcoreMesh(
        core_axis_name="c", subcore_axis_name="s",
        num_cores=NC, num_subcores=NS)

    @functools.partial(
        pl.kernel,
        out_type=(
            jax.ShapeDtypeStruct((NP_PAD, width), jnp.float32),
            jax.ShapeDtypeStruct((NP_PAD, width), jnp.float32),
            jax.ShapeDtypeStruct((NP_PAD, width), jnp.float32),
            jax.ShapeDtypeStruct((NP_PAD, width), jnp.float32),
            jax.ShapeDtypeStruct((NP_PAD, 16), jnp.float32),
        ),
        mesh=mesh,
        scratch_types=[
            pltpu.VMEM((CHUNK,), jnp.int32),
            pltpu.VMEM((CHUNK,), jnp.int32),
            pltpu.VMEM((CHUNK, width), jnp.float32),
            pltpu.VMEM((NPU + 32, ), jnp.int32),
            pltpu.VMEM((NPU, width), jnp.float32),
            pltpu.VMEM((NPU, width), jnp.float32),
            pltpu.VMEM((NPU, width), jnp.float32),
            pltpu.VMEM((NPU, width), jnp.float32),
            pltpu.VMEM((NPU, 16), jnp.float32),
            pltpu.SemaphoreType.DMA,
        ],
    )
    def k(h_hbm, src_hbm, dst_hbm, rp_hbm,
          o_s, o_q, o_mn, o_mx, o_c,
          idx_v, dst_v, msgs, rp_v, a_s, a_q, a_mn, a_mx, a_c, sem):
        wid = lax.axis_index("s") * NC + lax.axis_index("c")

        def _at(ref, i):
            return ref[pl.ds(i, 16)][0]
        zeros = jnp.zeros((16,), jnp.float32)
        pinf = jnp.full((16,), jnp.inf, jnp.float32)
        ninf = jnp.full((16,), -jnp.inf, jnp.float32)
        for ui in range(UPW):
            u = ui * NW + wid
            base = u * NPU
            pltpu.sync_copy(rp_hbm.at[pl.ds(base, NPU + 32)], rp_v)
            e0 = _at(rp_v, 0)
            e_end = _at(rp_v, NPU)

            def init_row(r, carry):
                for v in range(NF):
                    sl = pl.ds(v * 16, 16)
                    a_s[r, sl] = zeros
                    a_q[r, sl] = zeros
                    a_mn[r, sl] = pinf
                    a_mx[r, sl] = ninf
                a_c[r, pl.ds(0, 16)] = zeros
                return carry
            lax.fori_loop(0, NPU, init_row, 0)

            estart = e0 - lax.rem(e0, 8)
            nch = lax.div(e_end - estart + (CHUNK - 1), CHUNK)

            def chunk_body(c, carry0):
                e = pl.multiple_of(estart + c * CHUNK, 8)
                pltpu.sync_copy(src_hbm.at[pl.ds(e, CHUNK)], idx_v)
                pltpu.sync_copy(dst_hbm.at[pl.ds(e, CHUNK)], dst_v)
                pltpu.async_copy(h_hbm.at[idx_v], msgs, sem).wait()
                ghi = jnp.minimum(e_end, e + CHUNK)

                n_lo = jnp.maximum(dst_v[pl.ds(0, 16)][0] - base, 0)
                n_hi = jnp.minimum(
                    dst_v[pl.ds(CHUNK - 16, 16)][15] - base, NPU - 1)

                def proc_node(n, carry):
                    r0 = _at(rp_v, n)
                    r1 = _at(rp_v, n + 1)
                    lo = jnp.maximum(r0, e) - e
                    hi = jnp.minimum(r1, ghi) - e
                    for fc in range(FC):
                        sls = [pl.ds((fc * 8 + j) * 16, 16) for j in range(8)]
                        acc = ([a_s[n, s] for s in sls]
                               + [a_q[n, s] for s in sls]
                               + [a_mn[n, s] for s in sls]
                               + [a_mx[n, s] for s in sls])

                        def edge_body(i, ac):
                            m = [msgs[i, s] for s in sls]
                            return (
                                [ac[j] + m[j] for j in range(8)]
                                + [ac[8 + j] + m[j] * m[j] for j in range(8)]
                                + [jnp.minimum(ac[16 + j], m[j]) for j in range(8)]
                                + [jnp.maximum(ac[24 + j], m[j]) for j in range(8)]
                            )
                        acc = lax.fori_loop(lo, hi, edge_body, acc)
                        for j in range(8):
                            a_s[n, sls[j]] = acc[j]
                            a_q[n, sls[j]] = acc[8 + j]
                            a_mn[n, sls[j]] = acc[16 + j]
                            a_mx[n, sls[j]] = acc[24 + j]
                    dc = (hi - lo).astype(jnp.float32)
                    a_c[n, pl.ds(0, 16)] = a_c[n, pl.ds(0, 16)] + (zeros + dc)
                    return carry

                lax.fori_loop(n_lo, n_hi + 1, proc_node, 0)
                return carry0
            lax.fori_loop(0, nch, chunk_body, jnp.int32(0))

            pltpu.sync_copy(a_s, o_s.at[pl.ds(base, NPU)])
            pltpu.sync_copy(a_q, o_q.at[pl.ds(base, NPU)])
            pltpu.sync_copy(a_mn, o_mn.at[pl.ds(base, NPU)])
            pltpu.sync_copy(a_mx, o_mx.at[pl.ds(base, NPU)])
            pltpu.sync_copy(a_c, o_c.at[pl.ds(base, NPU)])
    return k


def _tc_layer_kernel(c_in, width):
    def body(s_ref, q_ref, mn_ref, mx_ref, c_ref, h_ref,
             wp_ref, bp_ref, wl_ref, bl_ref, wr_ref, br_ref,
             y_ref, ps_ref, pq_ref):
        cnt = c_ref[:, 0:1]
        denom = jnp.maximum(cnt, 1.0)
        mean = s_ref[:, :c_in] / denom
        mean2 = q_ref[:, :c_in] / denom
        var = mean2 - mean * mean
        std = jnp.sqrt(jnp.clip(var, 1e-5, None))
        has = cnt > 0.0
        mn = jnp.where(has, mn_ref[:, :c_in], 0.0)
        mx = jnp.where(has, mx_ref[:, :c_in], 0.0)
        wp = wp_ref[...]
        f32 = jnp.float32
        aggr = bp_ref[...]
        aggr = aggr + jnp.dot(mean, wp[0:c_in], preferred_element_type=f32)
        aggr = aggr + jnp.dot(mn, wp[c_in:2 * c_in], preferred_element_type=f32)
        aggr = aggr + jnp.dot(mx, wp[2 * c_in:3 * c_in], preferred_element_type=f32)
        aggr = aggr + jnp.dot(std, wp[3 * c_in:4 * c_in], preferred_element_type=f32)
        aggr = aggr + jnp.dot(var, wp[4 * c_in:5 * c_in], preferred_element_type=f32)
        out = (jnp.dot(aggr, wl_ref[...], preferred_element_type=f32) + bl_ref[...]
               + jnp.dot(h_ref[:, :c_in], wr_ref[...], preferred_element_type=f32)
               + br_ref[...])
        nrm = jnp.sqrt(jnp.sum(out * out, axis=1, keepdims=True))
        out = out / jnp.maximum(nrm, 1e-12)
        y = jnp.maximum(out, 0.0)
        y_ref[:, 0:H] = y
        y_ref[:, H:WY] = jnp.zeros((RB, WY - H), jnp.float32)
        ps_ref[0, 0, :] = jnp.sum(y, axis=0)
        pq_ref[0, 0, :] = jnp.sum(y * y, axis=0)

    stat_spec = pl.BlockSpec((RB, width), lambda b: (b, 0))
    full = lambda shape: pl.BlockSpec(shape, lambda b: (0, 0))
    return pl.pallas_call(
        body,
        grid=(NBLK,),
        in_specs=[
            stat_spec, stat_spec, stat_spec, stat_spec,
            pl.BlockSpec((RB, 16), lambda b: (b, 0)),
            pl.BlockSpec((RB, width), lambda b: (b, 0)),
            full((5 * c_in, H)), full((1, H)),
            full((H, H)), full((1, H)),
            full((c_in, H)), full((1, H)),
        ],
        out_specs=[
            pl.BlockSpec((RB, WY), lambda b: (b, 0)),
            pl.BlockSpec((1, 1, H), lambda b: (b, 0, 0)),
            pl.BlockSpec((1, 1, H), lambda b: (b, 0, 0)),
        ],
        out_shape=[
            jax.ShapeDtypeStruct((N, WY), jnp.float32),
            jax.ShapeDtypeStruct((NBLK, 1, H), jnp.float32),
            jax.ShapeDtypeStruct((NBLK, 1, H), jnp.float32),
        ],
    )


def _tc_bn_kernel():
    def body(y_ref, ps_ref, pq_ref, g_ref, b_ref, o_ref):
        mu = jnp.sum(ps_ref[...], axis=0) / N
        varb = jnp.sum(pq_ref[...], axis=0) / N - mu * mu
        scale = g_ref[...] / jnp.sqrt(varb + 1e-5)
        o_ref[:, 0:H] = (y_ref[:, 0:H] - mu) * scale + b_ref[...]
        o_ref[:, H:WY] = jnp.zeros((RB, WY - H), jnp.float32)

    full = lambda shape: pl.BlockSpec(shape, lambda b: (0, 0))
    return pl.pallas_call(
        body,
        grid=(NBLK,),
        in_specs=[
            pl.BlockSpec((RB, WY), lambda b: (b, 0)),
            pl.BlockSpec((NBLK, 1, H), lambda b: (0, 0, 0)),
            pl.BlockSpec((NBLK, 1, H), lambda b: (0, 0, 0)),
            full((1, H)), full((1, H)),
        ],
        out_specs=pl.BlockSpec((RB, WY), lambda b: (b, 0)),
        out_shape=jax.ShapeDtypeStruct((N, WY), jnp.float32),
    )


def kernel(x, edge_index, params):
    src = edge_index[0]
    dst = edge_index[1]
    order = jnp.argsort(dst)
    dst_s = dst[order]
    src_s = src[order]
    rowptr = jnp.searchsorted(
        dst_s, jnp.arange(NP_PAD + 32, dtype=jnp.int32).clip(0, N),
        side="left").astype(jnp.int32)
    srcp = jnp.concatenate([src_s, jnp.zeros((CHUNK + 8,), jnp.int32)])
    dstp = jnp.concatenate([dst_s, jnp.full((CHUNK + 8,), N, jnp.int32)])

    sc128 = _sc_stats_kernel(128)
    sc256 = _sc_stats_kernel(WY)
    bn = _tc_bn_kernel()

    h = jnp.pad(x, ((0, 0), (0, 125)))
    for i, p in enumerate(params):
        c_in = 3 if i == 0 else H
        width = 128 if i == 0 else WY
        s, q, mn, mx, cnt = (sc128 if i == 0 else sc256)(h, srcp, dstp, rowptr)
        k1 = _tc_layer_kernel(c_in, width)
        y, ps, pq = k1(
            s[:N], q[:N], mn[:N], mx[:N], cnt[:N], h,
            p["W_proj"], p["b_proj"].reshape(1, H),
            p["W_l"], p["b_l"].reshape(1, H),
            p["W_r"], p["b_r"].reshape(1, H))
        h = bn(y, ps, pq, p["gamma"].reshape(1, H), p["beta"].reshape(1, H))
    return h[:, :H]


# speedup vs baseline: 7.3979x; 1.0800x over previous
import functools

import jax
import jax.numpy as jnp
from jax import lax
from jax.experimental import pallas as pl
from jax.experimental.pallas import tpu as pltpu
from jax.experimental.pallas import tpu_sc as plsc

N = 10000
E = 320000
H = 200
NBLK = 10
RB = N // NBLK

NC = 2
NS = 16
NW = NC * NS
UNITS = 128
UPW = UNITS // NW
NPU = 80
WY = 256
NP_PAD = UNITS * NPU
CHUNK = 64


def _sc_stats_kernel(width):
    NF = width // 16
    FC = NF // 8
    mesh = plsc.VectorSubcoreMesh(
        core_axis_name="c", subcore_axis_name="s",
        num_cores=NC, num_subcores=NS)

    @functools.partial(
        pl.kernel,
        out_type=(
            jax.ShapeDtypeStruct((NP_PAD, width), jnp.float32),
            jax.ShapeDtypeStruct((NP_PAD, width), jnp.float32),
            jax.ShapeDtypeStruct((NP_PAD, width), jnp.float32),
            jax.ShapeDtypeStruct((NP_PAD, width), jnp.float32),
            jax.ShapeDtypeStruct((NP_PAD, 16), jnp.float32),
        ),
        mesh=mesh,
        scratch_types=[
            pltpu.VMEM((CHUNK,), jnp.int32),
            pltpu.VMEM((CHUNK,), jnp.int32),
            pltpu.VMEM((CHUNK, width), jnp.float32),
            pltpu.VMEM((CHUNK,), jnp.int32),
            pltpu.VMEM((CHUNK,), jnp.int32),
            pltpu.VMEM((CHUNK, width), jnp.float32),
            pltpu.VMEM((NPU + 32, ), jnp.int32),
            pltpu.VMEM((NPU, width), jnp.float32),
            pltpu.VMEM((NPU, width), jnp.float32),
            pltpu.VMEM((NPU, width), jnp.float32),
            pltpu.VMEM((NPU, width), jnp.float32),
            pltpu.VMEM((NPU, 16), jnp.float32),
            pltpu.SemaphoreType.DMA,
            pltpu.SemaphoreType.DMA,
        ],
    )
    def k(h_hbm, src_hbm, dst_hbm, rp_hbm,
          o_s, o_q, o_mn, o_mx, o_c,
          idx_a, dst_a, msgs_a, idx_b, dst_b, msgs_b,
          rp_v, a_s, a_q, a_mn, a_mx, a_c, sem_a, sem_b):
        wid = lax.axis_index("s") * NC + lax.axis_index("c")

        def _at(ref, i):
            return ref[pl.ds(i, 16)][0]
        zeros = jnp.zeros((16,), jnp.float32)
        pinf = jnp.full((16,), jnp.inf, jnp.float32)
        ninf = jnp.full((16,), -jnp.inf, jnp.float32)
        for ui in range(UPW):
            u = ui * NW + wid
            base = u * NPU
            pltpu.sync_copy(rp_hbm.at[pl.ds(base, NPU + 32)], rp_v)
            e0 = _at(rp_v, 0)
            e_end = _at(rp_v, NPU)

            def init_row(r, carry):
                for v in range(NF):
                    sl = pl.ds(v * 16, 16)
                    a_s[r, sl] = zeros
                    a_q[r, sl] = zeros
                    a_mn[r, sl] = pinf
                    a_mx[r, sl] = ninf
                a_c[r, pl.ds(0, 16)] = zeros
                return carry
            lax.fori_loop(0, NPU, init_row, 0)

            estart = e0 - lax.rem(e0, 8)
            nch = lax.div(e_end - estart + (CHUNK - 1), CHUNK)

            def fetch(idx_v, dst_v, msgs, sem, c):
                e = pl.multiple_of(estart + c * CHUNK, 8)
                pltpu.sync_copy(src_hbm.at[pl.ds(e, CHUNK)], idx_v)
                pltpu.sync_copy(dst_hbm.at[pl.ds(e, CHUNK)], dst_v)
                pltpu.async_copy(h_hbm.at[idx_v], msgs, sem)

            def process(dst_v, msgs, c):
                e = pl.multiple_of(estart + c * CHUNK, 8)
                ghi = jnp.minimum(e_end, e + CHUNK)

                n_hi = jnp.minimum(
                    dst_v[pl.ds(CHUNK - 16, 16)][15] - base, NPU - 1)
                n_lo = jnp.maximum(dst_v[pl.ds(0, 16)][0] - base, 0)
                n_lo = jnp.minimum(n_lo, n_hi + 1)

                def proc_node(n, carry):
                    r0 = _at(rp_v, n)
                    r1 = _at(rp_v, n + 1)
                    lo = jnp.maximum(r0, e) - e
                    hi = jnp.minimum(r1, ghi) - e
                    for fc in range(FC):
                        sls = [pl.ds((fc * 8 + j) * 16, 16) for j in range(8)]
                        acc = ([a_s[n, s] for s in sls]
                               + [a_q[n, s] for s in sls]
                               + [a_mn[n, s] for s in sls]
                               + [a_mx[n, s] for s in sls])

                        def edge_body(i, ac):
                            m = [msgs[i, s] for s in sls]
                            return (
                                [ac[j] + m[j] for j in range(8)]
                                + [ac[8 + j] + m[j] * m[j] for j in range(8)]
                                + [jnp.minimum(ac[16 + j], m[j]) for j in range(8)]
                                + [jnp.maximum(ac[24 + j], m[j]) for j in range(8)]
                            )
                        acc = lax.fori_loop(lo, hi, edge_body, acc)
                        for j in range(8):
                            a_s[n, sls[j]] = acc[j]
                            a_q[n, sls[j]] = acc[8 + j]
                            a_mn[n, sls[j]] = acc[16 + j]
                            a_mx[n, sls[j]] = acc[24 + j]
                    dc = (hi - lo).astype(jnp.float32)
                    a_c[n, pl.ds(0, 16)] = a_c[n, pl.ds(0, 16)] + (zeros + dc)
                    return carry

                lax.fori_loop(n_lo, n_hi + 1, proc_node, 0)

            fetch(idx_a, dst_a, msgs_a, sem_a, jnp.int32(0))
            npair = lax.div(nch + 1, 2)

            def pair_body(cp, carry0):
                c0 = 2 * cp
                fetch(idx_b, dst_b, msgs_b, sem_b, c0 + 1)
                pltpu.make_async_copy(h_hbm.at[idx_a], msgs_a, sem_a).wait()
                process(dst_a, msgs_a, c0)
                fetch(idx_a, dst_a, msgs_a, sem_a, c0 + 2)
                pltpu.make_async_copy(h_hbm.at[idx_b], msgs_b, sem_b).wait()
                process(dst_b, msgs_b, c0 + 1)
                return carry0
            lax.fori_loop(0, npair, pair_body, jnp.int32(0))
            pltpu.make_async_copy(h_hbm.at[idx_a], msgs_a, sem_a).wait()

            pltpu.sync_copy(a_s, o_s.at[pl.ds(base, NPU)])
            pltpu.sync_copy(a_q, o_q.at[pl.ds(base, NPU)])
            pltpu.sync_copy(a_mn, o_mn.at[pl.ds(base, NPU)])
            pltpu.sync_copy(a_mx, o_mx.at[pl.ds(base, NPU)])
            pltpu.sync_copy(a_c, o_c.at[pl.ds(base, NPU)])
    return k


def _tc_layer_kernel(c_in, width):
    def body(s_ref, q_ref, mn_ref, mx_ref, c_ref, h_ref,
             wp_ref, bp_ref, wl_ref, bl_ref, wr_ref, br_ref,
             y_ref, ps_ref, pq_ref):
        cnt = c_ref[:, 0:1]
        denom = jnp.maximum(cnt, 1.0)
        mean = s_ref[:, :c_in] / denom
        mean2 = q_ref[:, :c_in] / denom
        var = mean2 - mean * mean
        std = jnp.sqrt(jnp.clip(var, 1e-5, None))
        has = cnt > 0.0
        mn = jnp.where(has, mn_ref[:, :c_in], 0.0)
        mx = jnp.where(has, mx_ref[:, :c_in], 0.0)
        wp = wp_ref[...]
        f32 = jnp.float32
        aggr = bp_ref[...]
        aggr = aggr + jnp.dot(mean, wp[0:c_in], preferred_element_type=f32)
        aggr = aggr + jnp.dot(mn, wp[c_in:2 * c_in], preferred_element_type=f32)
        aggr = aggr + jnp.dot(mx, wp[2 * c_in:3 * c_in], preferred_element_type=f32)
        aggr = aggr + jnp.dot(std, wp[3 * c_in:4 * c_in], preferred_element_type=f32)
        aggr = aggr + jnp.dot(var, wp[4 * c_in:5 * c_in], preferred_element_type=f32)
        out = (jnp.dot(aggr, wl_ref[...], preferred_element_type=f32) + bl_ref[...]
               + jnp.dot(h_ref[:, :c_in], wr_ref[...], preferred_element_type=f32)
               + br_ref[...])
        nrm = jnp.sqrt(jnp.sum(out * out, axis=1, keepdims=True))
        out = out / jnp.maximum(nrm, 1e-12)
        y = jnp.maximum(out, 0.0)
        y_ref[:, 0:H] = y
        y_ref[:, H:WY] = jnp.zeros((RB, WY - H), jnp.float32)
        ps_ref[0, 0, :] = jnp.sum(y, axis=0)
        pq_ref[0, 0, :] = jnp.sum(y * y, axis=0)

    stat_spec = pl.BlockSpec((RB, width), lambda b: (b, 0))
    full = lambda shape: pl.BlockSpec(shape, lambda b: (0, 0))
    return pl.pallas_call(
        body,
        grid=(NBLK,),
        in_specs=[
            stat_spec, stat_spec, stat_spec, stat_spec,
            pl.BlockSpec((RB, 16), lambda b: (b, 0)),
            pl.BlockSpec((RB, width), lambda b: (b, 0)),
            full((5 * c_in, H)), full((1, H)),
            full((H, H)), full((1, H)),
            full((c_in, H)), full((1, H)),
        ],
        out_specs=[
            pl.BlockSpec((RB, WY), lambda b: (b, 0)),
            pl.BlockSpec((1, 1, H), lambda b: (b, 0, 0)),
            pl.BlockSpec((1, 1, H), lambda b: (b, 0, 0)),
        ],
        out_shape=[
            jax.ShapeDtypeStruct((N, WY), jnp.float32),
            jax.ShapeDtypeStruct((NBLK, 1, H), jnp.float32),
            jax.ShapeDtypeStruct((NBLK, 1, H), jnp.float32),
        ],
    )


def _tc_bn_kernel():
    def body(y_ref, ps_ref, pq_ref, g_ref, b_ref, o_ref):
        mu = jnp.sum(ps_ref[...], axis=0) / N
        varb = jnp.sum(pq_ref[...], axis=0) / N - mu * mu
        scale = g_ref[...] / jnp.sqrt(varb + 1e-5)
        o_ref[:, 0:H] = (y_ref[:, 0:H] - mu) * scale + b_ref[...]
        o_ref[:, H:WY] = jnp.zeros((RB, WY - H), jnp.float32)

    full = lambda shape: pl.BlockSpec(shape, lambda b: (0, 0))
    return pl.pallas_call(
        body,
        grid=(NBLK,),
        in_specs=[
            pl.BlockSpec((RB, WY), lambda b: (b, 0)),
            pl.BlockSpec((NBLK, 1, H), lambda b: (0, 0, 0)),
            pl.BlockSpec((NBLK, 1, H), lambda b: (0, 0, 0)),
            full((1, H)), full((1, H)),
        ],
        out_specs=pl.BlockSpec((RB, WY), lambda b: (b, 0)),
        out_shape=jax.ShapeDtypeStruct((N, WY), jnp.float32),
    )


def kernel(x, edge_index, params):
    src = edge_index[0]
    dst = edge_index[1]
    order = jnp.argsort(dst)
    dst_s = dst[order]
    src_s = src[order]
    rowptr = jnp.searchsorted(
        dst_s, jnp.arange(NP_PAD + 32, dtype=jnp.int32).clip(0, N),
        side="left").astype(jnp.int32)
    srcp = jnp.concatenate([src_s, jnp.zeros((3 * CHUNK + 8,), jnp.int32)])
    dstp = jnp.concatenate([dst_s, jnp.full((3 * CHUNK + 8,), N, jnp.int32)])

    sc128 = _sc_stats_kernel(128)
    sc256 = _sc_stats_kernel(WY)
    bn = _tc_bn_kernel()

    h = jnp.pad(x, ((0, 0), (0, 125)))
    for i, p in enumerate(params):
        c_in = 3 if i == 0 else H
        width = 128 if i == 0 else WY
        s, q, mn, mx, cnt = (sc128 if i == 0 else sc256)(h, srcp, dstp, rowptr)
        k1 = _tc_layer_kernel(c_in, width)
        y, ps, pq = k1(
            s[:N], q[:N], mn[:N], mx[:N], cnt[:N], h,
            p["W_proj"], p["b_proj"].reshape(1, H),
            p["W_l"], p["b_l"].reshape(1, H),
            p["W_r"], p["b_r"].reshape(1, H))
        h = bn(y, ps, pq, p["gamma"].reshape(1, H), p["beta"].reshape(1, H))
    return h[:, :H]


# speedup vs baseline: 7.6776x; 1.0378x over previous
import functools

import jax
import jax.numpy as jnp
from jax import lax
from jax.experimental import pallas as pl
from jax.experimental.pallas import tpu as pltpu
from jax.experimental.pallas import tpu_sc as plsc

N = 10000
E = 320000
H = 200
NBLK = 10
RB = 1024

NC = 2
NS = 16
NW = NC * NS
UNITS = 128
UPW = UNITS // NW
NPU = 80
WY = 256
NP_PAD = UNITS * NPU
CHUNK = 64


def _sc_stats_kernel(width):
    NF = width // 16
    FC = NF // 8
    mesh = plsc.VectorSubcoreMesh(
        core_axis_name="c", subcore_axis_name="s",
        num_cores=NC, num_subcores=NS)

    @functools.partial(
        pl.kernel,
        out_type=(
            jax.ShapeDtypeStruct((NP_PAD, width), jnp.float32),
            jax.ShapeDtypeStruct((NP_PAD, width), jnp.float32),
            jax.ShapeDtypeStruct((NP_PAD, width), jnp.float32),
            jax.ShapeDtypeStruct((NP_PAD, width), jnp.float32),
            jax.ShapeDtypeStruct((NP_PAD, 16), jnp.float32),
        ),
        mesh=mesh,
        scratch_types=[
            pltpu.VMEM((CHUNK,), jnp.int32),
            pltpu.VMEM((CHUNK,), jnp.int32),
            pltpu.VMEM((CHUNK, width), jnp.float32),
            pltpu.VMEM((CHUNK,), jnp.int32),
            pltpu.VMEM((CHUNK,), jnp.int32),
            pltpu.VMEM((CHUNK, width), jnp.float32),
            pltpu.VMEM((NPU + 32, ), jnp.int32),
            pltpu.VMEM((NPU, width), jnp.float32),
            pltpu.VMEM((NPU, width), jnp.float32),
            pltpu.VMEM((NPU, width), jnp.float32),
            pltpu.VMEM((NPU, width), jnp.float32),
            pltpu.VMEM((NPU, 16), jnp.float32),
            pltpu.SemaphoreType.DMA,
            pltpu.SemaphoreType.DMA,
        ],
    )
    def k(h_hbm, src_hbm, dst_hbm, rp_hbm,
          o_s, o_q, o_mn, o_mx, o_c,
          idx_a, dst_a, msgs_a, idx_b, dst_b, msgs_b,
          rp_v, a_s, a_q, a_mn, a_mx, a_c, sem_a, sem_b):
        wid = lax.axis_index("s") * NC + lax.axis_index("c")

        def _at(ref, i):
            return ref[pl.ds(i, 16)][0]
        zeros = jnp.zeros((16,), jnp.float32)
        pinf = jnp.full((16,), jnp.inf, jnp.float32)
        ninf = jnp.full((16,), -jnp.inf, jnp.float32)
        for ui in range(UPW):
            u = ui * NW + wid
            base = u * NPU
            pltpu.sync_copy(rp_hbm.at[pl.ds(base, NPU + 32)], rp_v)
            e0 = _at(rp_v, 0)
            e_end = _at(rp_v, NPU)

            def init_row(r, carry):
                for v in range(NF):
                    sl = pl.ds(v * 16, 16)
                    a_s[r, sl] = zeros
                    a_q[r, sl] = zeros
                    a_mn[r, sl] = pinf
                    a_mx[r, sl] = ninf
                a_c[r, pl.ds(0, 16)] = zeros
                return carry
            lax.fori_loop(0, NPU, init_row, 0)

            estart = e0 - lax.rem(e0, 8)
            nch = lax.div(e_end - estart + (CHUNK - 1), CHUNK)

            def fetch(idx_v, dst_v, msgs, sem, c):
                e = pl.multiple_of(estart + c * CHUNK, 8)
                pltpu.sync_copy(src_hbm.at[pl.ds(e, CHUNK)], idx_v)
                pltpu.sync_copy(dst_hbm.at[pl.ds(e, CHUNK)], dst_v)
                pltpu.async_copy(h_hbm.at[idx_v], msgs, sem)

            def process(dst_v, msgs, c):
                e = pl.multiple_of(estart + c * CHUNK, 8)
                ghi = jnp.minimum(e_end, e + CHUNK)

                n_hi = jnp.minimum(
                    dst_v[pl.ds(CHUNK - 16, 16)][15] - base, NPU - 1)
                n_lo = jnp.maximum(dst_v[pl.ds(0, 16)][0] - base, 0)
                n_lo = jnp.minimum(n_lo, n_hi + 1)

                def proc_node(n, carry):
                    r0 = _at(rp_v, n)
                    r1 = _at(rp_v, n + 1)
                    lo = jnp.maximum(r0, e) - e
                    hi = jnp.minimum(r1, ghi) - e
                    for fc in range(FC):
                        sls = [pl.ds((fc * 8 + j) * 16, 16) for j in range(8)]
                        acc = ([a_s[n, s] for s in sls]
                               + [a_q[n, s] for s in sls]
                               + [a_mn[n, s] for s in sls]
                               + [a_mx[n, s] for s in sls])

                        def edge_body(i, ac):
                            m = [msgs[i, s] for s in sls]
                            return (
                                [ac[j] + m[j] for j in range(8)]
                                + [ac[8 + j] + m[j] * m[j] for j in range(8)]
                                + [jnp.minimum(ac[16 + j], m[j]) for j in range(8)]
                                + [jnp.maximum(ac[24 + j], m[j]) for j in range(8)]
                            )
                        acc = lax.fori_loop(lo, hi, edge_body, acc)
                        for j in range(8):
                            a_s[n, sls[j]] = acc[j]
                            a_q[n, sls[j]] = acc[8 + j]
                            a_mn[n, sls[j]] = acc[16 + j]
                            a_mx[n, sls[j]] = acc[24 + j]
                    dc = (hi - lo).astype(jnp.float32)
                    a_c[n, pl.ds(0, 16)] = a_c[n, pl.ds(0, 16)] + (zeros + dc)
                    return carry

                lax.fori_loop(n_lo, n_hi + 1, proc_node, 0)

            fetch(idx_a, dst_a, msgs_a, sem_a, jnp.int32(0))
            npair = lax.div(nch + 1, 2)

            def pair_body(cp, carry0):
                c0 = 2 * cp
                fetch(idx_b, dst_b, msgs_b, sem_b, c0 + 1)
                pltpu.make_async_copy(h_hbm.at[idx_a], msgs_a, sem_a).wait()
                process(dst_a, msgs_a, c0)
                fetch(idx_a, dst_a, msgs_a, sem_a, c0 + 2)
                pltpu.make_async_copy(h_hbm.at[idx_b], msgs_b, sem_b).wait()
                process(dst_b, msgs_b, c0 + 1)
                return carry0
            lax.fori_loop(0, npair, pair_body, jnp.int32(0))
            pltpu.make_async_copy(h_hbm.at[idx_a], msgs_a, sem_a).wait()

            pltpu.sync_copy(a_s, o_s.at[pl.ds(base, NPU)])
            pltpu.sync_copy(a_q, o_q.at[pl.ds(base, NPU)])
            pltpu.sync_copy(a_mn, o_mn.at[pl.ds(base, NPU)])
            pltpu.sync_copy(a_mx, o_mx.at[pl.ds(base, NPU)])
            pltpu.sync_copy(a_c, o_c.at[pl.ds(base, NPU)])
    return k


def _tc_layer_kernel(c_in, width):
    def body(s_ref, q_ref, mn_ref, mx_ref, c_ref, h_ref,
             wp_ref, bp_ref, wl_ref, bl_ref, wr_ref, br_ref,
             y_ref, ps_ref, pq_ref):
        cnt = c_ref[:, 0:1]
        denom = jnp.maximum(cnt, 1.0)
        mean = s_ref[:, :c_in] / denom
        mean2 = q_ref[:, :c_in] / denom
        var = mean2 - mean * mean
        std = jnp.sqrt(jnp.clip(var, 1e-5, None))
        has = cnt > 0.0
        mn = jnp.where(has, mn_ref[:, :c_in], 0.0)
        mx = jnp.where(has, mx_ref[:, :c_in], 0.0)
        wp = wp_ref[...]
        f32 = jnp.float32
        aggr = bp_ref[...]
        aggr = aggr + jnp.dot(mean, wp[0:c_in], preferred_element_type=f32)
        aggr = aggr + jnp.dot(mn, wp[c_in:2 * c_in], preferred_element_type=f32)
        aggr = aggr + jnp.dot(mx, wp[2 * c_in:3 * c_in], preferred_element_type=f32)
        aggr = aggr + jnp.dot(std, wp[3 * c_in:4 * c_in], preferred_element_type=f32)
        aggr = aggr + jnp.dot(var, wp[4 * c_in:5 * c_in], preferred_element_type=f32)
        out = (jnp.dot(aggr, wl_ref[...], preferred_element_type=f32) + bl_ref[...]
               + jnp.dot(h_ref[:, :c_in], wr_ref[...], preferred_element_type=f32)
               + br_ref[...])
        nrm = jnp.sqrt(jnp.sum(out * out, axis=1, keepdims=True))
        out = out / jnp.maximum(nrm, 1e-12)
        y = jnp.maximum(out, 0.0)
        y_ref[:, 0:H] = y
        y_ref[:, H:WY] = jnp.zeros((RB, WY - H), jnp.float32)
        rowid = (lax.broadcasted_iota(jnp.int32, (RB, 1), 0)
                 + pl.program_id(0) * RB)
        ym = jnp.where(rowid < N, y, 0.0)
        ps_ref[0, 0, :] = jnp.sum(ym, axis=0)
        pq_ref[0, 0, :] = jnp.sum(ym * ym, axis=0)

    stat_spec = pl.BlockSpec((RB, width), lambda b: (b, 0))
    full = lambda shape: pl.BlockSpec(shape, lambda b: (0, 0))
    return pl.pallas_call(
        body,
        grid=(NBLK,),
        in_specs=[
            stat_spec, stat_spec, stat_spec, stat_spec,
            pl.BlockSpec((RB, 16), lambda b: (b, 0)),
            pl.BlockSpec((RB, width), lambda b: (b, 0)),
            full((5 * c_in, H)), full((1, H)),
            full((H, H)), full((1, H)),
            full((c_in, H)), full((1, H)),
        ],
        out_specs=[
            pl.BlockSpec((RB, WY), lambda b: (b, 0)),
            pl.BlockSpec((1, 1, H), lambda b: (b, 0, 0)),
            pl.BlockSpec((1, 1, H), lambda b: (b, 0, 0)),
        ],
        out_shape=[
            jax.ShapeDtypeStruct((NP_PAD, WY), jnp.float32),
            jax.ShapeDtypeStruct((NBLK, 1, H), jnp.float32),
            jax.ShapeDtypeStruct((NBLK, 1, H), jnp.float32),
        ],
    )


def _tc_bn_kernel():
    def body(y_ref, ps_ref, pq_ref, g_ref, b_ref, o_ref):
        mu = jnp.sum(ps_ref[...], axis=0) / N
        varb = jnp.sum(pq_ref[...], axis=0) / N - mu * mu
        scale = g_ref[...] / jnp.sqrt(varb + 1e-5)
        o_ref[:, 0:H] = (y_ref[:, 0:H] - mu) * scale + b_ref[...]
        o_ref[:, H:WY] = jnp.zeros((RB, WY - H), jnp.float32)

    full = lambda shape: pl.BlockSpec(shape, lambda b: (0, 0))
    return pl.pallas_call(
        body,
        grid=(NBLK,),
        in_specs=[
            pl.BlockSpec((RB, WY), lambda b: (b, 0)),
            pl.BlockSpec((NBLK, 1, H), lambda b: (0, 0, 0)),
            pl.BlockSpec((NBLK, 1, H), lambda b: (0, 0, 0)),
            full((1, H)), full((1, H)),
        ],
        out_specs=pl.BlockSpec((RB, WY), lambda b: (b, 0)),
        out_shape=jax.ShapeDtypeStruct((NP_PAD, WY), jnp.float32),
    )


def kernel(x, edge_index, params):
    order = jnp.argsort(edge_index[1])
    ei_s = edge_index[:, order]
    src_s = ei_s[0]
    dst_s = ei_s[1]
    rowptr = jnp.searchsorted(
        dst_s, jnp.arange(NP_PAD + 32, dtype=jnp.int32).clip(0, N),
        side="left").astype(jnp.int32)
    srcp = jnp.concatenate([src_s, jnp.zeros((3 * CHUNK + 8,), jnp.int32)])
    dstp = jnp.concatenate([dst_s, jnp.full((3 * CHUNK + 8,), N, jnp.int32)])

    sc128 = _sc_stats_kernel(128)
    sc256 = _sc_stats_kernel(WY)
    bn = _tc_bn_kernel()

    h = jnp.pad(x, ((0, NP_PAD - N), (0, 125)))
    for i, p in enumerate(params):
        c_in = 3 if i == 0 else H
        width = 128 if i == 0 else WY
        s, q, mn, mx, cnt = (sc128 if i == 0 else sc256)(h, srcp, dstp, rowptr)
        k1 = _tc_layer_kernel(c_in, width)
        y, ps, pq = k1(
            s, q, mn, mx, cnt, h,
            p["W_proj"], p["b_proj"].reshape(1, H),
            p["W_l"], p["b_l"].reshape(1, H),
            p["W_r"], p["b_r"].reshape(1, H))
        h = bn(y, ps, pq, p["gamma"].reshape(1, H), p["beta"].reshape(1, H))
    return h[:N, :H]
